# Initial kernel scaffold; baseline (speedup 1.0000x reference)
import jax
import jax.numpy as jnp
from jax.experimental import pallas as pl


def kernel(x, edge_index, params, eps):
    raise NotImplementedError("write your pallas kernel here")


# speedup vs baseline: 13.6459x; 13.6459x over previous
import functools

import jax
import jax.numpy as jnp
from jax import lax
from jax.experimental import pallas as pl
from jax.experimental.pallas import tpu as pltpu
from jax.experimental.pallas import tpu_sc as plsc

N = 10000
E = 320000
D_FEAT = 128
H1 = 32
H2 = 16
NUM_SUM = 100
NUM_LEAF = 20
Y_DIMS = 16

NC = 2
NS = 16
NW = NC * NS
CHUNK = 125
ROWS = E // CHUNK
ROWS_W = ROWS // NW
STRIPE = N // NS

S_BIG = 632
S_LAST = N - (NS - 1) * S_BIG

W_P0 = 128
W_P1 = 128
W_DEG = 8

RECON_BLK = 400
TC_BLK = 1000
OUT_WORDS = N * N + N * NUM_SUM + N * Y_DIMS * NUM_LEAF


def _sc_mesh():
    return plsc.VectorSubcoreMesh(core_axis_name="c", subcore_axis_name="s")


def _stripe_zero(zer_hbm, acc, s):
    @pl.when(s < NS - 1)
    def _():
        pltpu.sync_copy(zer_hbm, acc.at[pl.ds(s * S_BIG, S_BIG)])

    @pl.when(s == NS - 1)
    def _():
        pltpu.sync_copy(zer_hbm.at[pl.ds(0, S_LAST)],
                        acc.at[pl.ds((NS - 1) * S_BIG, S_LAST)])


def _stripe_out(acc, out_hbm, c, s):
    @pl.when(s < NS - 1)
    def _():
        pltpu.sync_copy(acc.at[pl.ds(s * S_BIG, S_BIG)],
                        out_hbm.at[c, pl.ds(s * S_BIG, S_BIG)])

    @pl.when(s == NS - 1)
    def _():
        pltpu.sync_copy(acc.at[pl.ds((NS - 1) * S_BIG, S_LAST)],
                        out_hbm.at[c, pl.ds((NS - 1) * S_BIG, S_LAST)])


def _deg_kernel(dst_hbm, ones_hbm, zer_hbm, out_hbm, dst_v, ones_v, acc):
    c = lax.axis_index("c")
    s = lax.axis_index("s")
    wid = c * NS + s
    _stripe_zero(zer_hbm, acc, s)
    pltpu.sync_copy(dst_hbm.at[pl.ds(wid * ROWS_W, ROWS_W)], dst_v)
    pltpu.sync_copy(ones_hbm, ones_v)
    plsc.subcore_barrier()

    def body(j, carry):
        pltpu.sync_copy(ones_v, acc.at[dst_v.at[j]], add=True)
        return carry

    lax.fori_loop(0, ROWS_W, body, 0)
    plsc.subcore_barrier()
    _stripe_out(acc, out_hbm, c, s)


def _make_deg():
    return functools.partial(
        pl.kernel,
        mesh=_sc_mesh(),
        out_type=jax.ShapeDtypeStruct((NC, N, W_DEG), jnp.float32),
        scratch_types=[
            pltpu.VMEM((ROWS_W, CHUNK), jnp.int32),
            pltpu.VMEM((CHUNK, W_DEG), jnp.float32),
            pltpu.VMEM_SHARED((N, W_DEG), jnp.float32),
        ],
    )(_deg_kernel)


def _prop_kernel(src_hbm, dst_hbm, tbl_hbm, zer_hbm, out_hbm,
                 src_v, dst_v, rows_v, acc, sem):
    c = lax.axis_index("c")
    s = lax.axis_index("s")
    wid = c * NS + s
    _stripe_zero(zer_hbm, acc, s)
    pltpu.sync_copy(src_hbm.at[pl.ds(wid * ROWS_W, ROWS_W)], src_v)
    pltpu.sync_copy(dst_hbm.at[pl.ds(wid * ROWS_W, ROWS_W)], dst_v)
    plsc.subcore_barrier()

    def body(j, carry):
        pltpu.async_copy(tbl_hbm.at[src_v.at[j]], rows_v, sem).wait()
        pltpu.sync_copy(rows_v, acc.at[dst_v.at[j]], add=True)
        return carry

    lax.fori_loop(0, ROWS_W, body, 0)
    plsc.subcore_barrier()
    _stripe_out(acc, out_hbm, c, s)


def _make_prop(width):
    return functools.partial(
        pl.kernel,
        mesh=_sc_mesh(),
        out_type=jax.ShapeDtypeStruct((NC, N, width), jnp.float32),
        scratch_types=[
            pltpu.VMEM((ROWS_W, CHUNK), jnp.int32),
            pltpu.VMEM((ROWS_W, CHUNK), jnp.int32),
            pltpu.VMEM((CHUNK, width), jnp.float32),
            pltpu.VMEM_SHARED((N, width), jnp.float32),
            pltpu.SemaphoreType.DMA,
        ],
    )(_prop_kernel)


def _prep_body(deg_ref, x_ref, xs_ref, dinv_ref):
    deg = deg_ref[0, :, 0:1] + deg_ref[1, :, 0:1] + 1.0
    dv = lax.rsqrt(deg)
    dinv_ref[...] = dv
    xs_ref[...] = x_ref[...] * dv


def _mid_body(p_ref, dinv_ref, wg_ref, wb_ref, out_ref):
    dv = dinv_ref[...]
    agg = (p_ref[0] + p_ref[1]) * dv
    h = jnp.maximum(
        jnp.dot(agg, wg_ref[...], preferred_element_type=jnp.float32), 0.0)
    g = jnp.dot(h, wb_ref[...], preferred_element_type=jnp.float32)
    out_ref[...] = g * dv


def _zprep_body(p_ref, dinv_ref, eps_ref, z_ref, zm_ref):
    dv = dinv_ref[...]
    p = (p_ref[0] + p_ref[1]) * dv
    zm = p[:, 0:H2]
    acc = zm
    for i in range(5):
        acc = acc + eps_ref[i] * jnp.exp(p[:, H2 + H2 * i:2 * H2 + H2 * i])
    z_ref[...] = 0.2 * acc
    zm_ref[...] = zm


def _recon_body(z_ref, zt_ref, out_ref):
    out_ref[...] = jnp.dot(z_ref[...], zt_ref[...],
                           preferred_element_type=jnp.float32)


def _sw_body(zm_ref, wsw_ref, bsw_ref, tail_ref, out_ref, buf, sem):
    del tail_ref
    i = pl.program_id(0)
    buf[...] = (
        jnp.dot(zm_ref[...], wsw_ref[...], preferred_element_type=jnp.float32)
        + bsw_ref[...])
    cp = pltpu.make_async_copy(
        buf, out_ref.at[pl.ds(N * N // NUM_SUM + i * TC_BLK, TC_BLK)], sem)
    cp.start()
    cp.wait()


def _leaf_body(zm_ref, wl1_ref, bl1_ref, wl2_ref, bl2_ref, wlw_ref, blw_ref,
               out_ref):
    l1 = jnp.maximum(
        jnp.dot(zm_ref[...], wl1_ref[...], preferred_element_type=jnp.float32)
        + bl1_ref[...], 0.0)
    l2 = jnp.maximum(
        jnp.dot(l1, wl2_ref[...], preferred_element_type=jnp.float32)
        + bl2_ref[...], 0.0)
    for j in range(Y_DIMS):
        out_ref[:, pl.ds(j * NUM_LEAF, NUM_LEAF)] = (
            jnp.dot(l2[:, j * H2:(j + 1) * H2], wlw_ref[...],
                    preferred_element_type=jnp.float32) + blw_ref[...])


def kernel(x, edge_index, params, eps):
    f32 = jnp.float32
    src2 = edge_index[0].reshape(ROWS, CHUNK)
    dst2 = edge_index[1].reshape(ROWS, CHUNK)
    ones8 = jnp.ones((CHUNK, W_DEG), f32)
    zer8 = jnp.zeros((S_BIG, W_DEG), f32)
    zer128 = jnp.zeros((S_BIG, W_P0), f32)

    wg = jnp.concatenate([params['W_gc%d' % i] for i in range(5)], axis=1)
    wm = jnp.concatenate([params['W_mean%d' % i] for i in range(5)], axis=0)
    ws = jnp.zeros((5 * H1, 5 * H2), f32)
    for i in range(5):
        ws = ws.at[i * H1:(i + 1) * H1, i * H2:(i + 1) * H2].set(
            params['W_std%d' % i])
    wb = jnp.concatenate([wm, ws, jnp.zeros((5 * H1, 32), f32)], axis=1)
    wsw = params['W_sw1'] @ params['W_sw2']
    bsw = params['b_sw1'] @ params['W_sw2'] + params['b_sw2']

    deg8 = _make_deg()(dst2, ones8, zer8)

    xs, dinv = pl.pallas_call(
        _prep_body,
        grid=(N // TC_BLK,),
        in_specs=[
            pl.BlockSpec((NC, TC_BLK, W_DEG), lambda i: (0, i, 0)),
            pl.BlockSpec((TC_BLK, D_FEAT), lambda i: (i, 0)),
        ],
        out_specs=(pl.BlockSpec((TC_BLK, D_FEAT), lambda i: (i, 0)),
                   pl.BlockSpec((TC_BLK, 1), lambda i: (i, 0))),
        out_shape=(jax.ShapeDtypeStruct((N, D_FEAT), f32),
                   jax.ShapeDtypeStruct((N, 1), f32)),
    )(deg8, x)

    p0 = _make_prop(W_P0)(src2, dst2, xs, zer128)

    gs = pl.pallas_call(
        _mid_body,
        grid=(N // TC_BLK,),
        in_specs=[
            pl.BlockSpec((NC, TC_BLK, W_P0), lambda i: (0, i, 0)),
            pl.BlockSpec((TC_BLK, 1), lambda i: (i, 0)),
            pl.BlockSpec((D_FEAT, 5 * H1), lambda i: (0, 0)),
            pl.BlockSpec((5 * H1, W_P1), lambda i: (0, 0)),
        ],
        out_specs=pl.BlockSpec((TC_BLK, W_P1), lambda i: (i, 0)),
        out_shape=jax.ShapeDtypeStruct((N, W_P1), f32),
    )(p0, dinv, wg, wb)

    p1 = _make_prop(W_P1)(src2, dst2, gs, zer128)

    z, zm = pl.pallas_call(
        _zprep_body,
        grid=(N // TC_BLK,),
        in_specs=[
            pl.BlockSpec((NC, TC_BLK, W_P1), lambda i: (0, i, 0)),
            pl.BlockSpec((TC_BLK, 1), lambda i: (i, 0)),
            pl.BlockSpec((5, TC_BLK, H2), lambda i: (0, i, 0)),
        ],
        out_specs=(pl.BlockSpec((TC_BLK, H2), lambda i: (i, 0)),
                   pl.BlockSpec((TC_BLK, H2), lambda i: (i, 0))),
        out_shape=(jax.ShapeDtypeStruct((N, H2), f32),
                   jax.ShapeDtypeStruct((N, H2), f32)),
    )(p1, dinv, eps)

    zt = z.T

    flat2d = pl.pallas_call(
        _recon_body,
        grid=(N // RECON_BLK,),
        in_specs=[
            pl.BlockSpec((RECON_BLK, H2), lambda i: (i, 0)),
            pl.BlockSpec((H2, N), lambda i: (0, 0)),
        ],
        out_specs=pl.BlockSpec((RECON_BLK, N), lambda i: (i, 0)),
        out_shape=jax.ShapeDtypeStruct((OUT_WORDS // N, N), f32),
    )(z, zt)

    v_sw = flat2d.reshape(OUT_WORDS // NUM_SUM, NUM_SUM)
    v_sw = pl.pallas_call(
        _sw_body,
        grid=(N // TC_BLK,),
        in_specs=[
            pl.BlockSpec((TC_BLK, H2), lambda i: (i, 0)),
            pl.BlockSpec((H2, NUM_SUM), lambda i: (0, 0)),
            pl.BlockSpec((1, NUM_SUM), lambda i: (0, 0)),
            pl.BlockSpec(memory_space=pl.ANY),
        ],
        out_specs=pl.BlockSpec(memory_space=pl.ANY),
        out_shape=jax.ShapeDtypeStruct(v_sw.shape, f32),
        scratch_shapes=[pltpu.VMEM((TC_BLK, NUM_SUM), f32),
                        pltpu.SemaphoreType.DMA],
        input_output_aliases={3: 0},
    )(zm, wsw, bsw.reshape(1, NUM_SUM), v_sw)

    leaf = pl.pallas_call(
        _leaf_body,
        grid=(N // TC_BLK,),
        in_specs=[
            pl.BlockSpec((TC_BLK, H2), lambda i: (i, 0)),
            pl.BlockSpec((H2, Y_DIMS * H2), lambda i: (0, 0)),
            pl.BlockSpec((1, Y_DIMS * H2), lambda i: (0, 0)),
            pl.BlockSpec((Y_DIMS * H2, Y_DIMS * H2), lambda i: (0, 0)),
            pl.BlockSpec((1, Y_DIMS * H2), lambda i: (0, 0)),
            pl.BlockSpec((H2, NUM_LEAF), lambda i: (0, 0)),
            pl.BlockSpec((1, NUM_LEAF), lambda i: (0, 0)),
        ],
        out_specs=pl.BlockSpec((TC_BLK, Y_DIMS * NUM_LEAF), lambda i: (i, 0)),
        out_shape=jax.ShapeDtypeStruct((N, Y_DIMS * NUM_LEAF), f32),
    )(zm, params['W_l1'], params['b_l1'].reshape(1, -1),
      params['W_l2'], params['b_l2'].reshape(1, -1),
      params['W_lw'], params['b_lw'].reshape(1, NUM_LEAF))

    flat = v_sw.reshape(-1)
    flat = lax.dynamic_update_slice(
        flat, leaf.reshape(-1), (N * N + N * NUM_SUM,))
    return flat


# speedup vs baseline: 25.1849x; 1.8456x over previous
import functools

import jax
import jax.numpy as jnp
from jax import lax
from jax.experimental import pallas as pl
from jax.experimental.pallas import tpu as pltpu
from jax.experimental.pallas import tpu_sc as plsc

N = 10000
E = 320000
D_FEAT = 128
H1 = 32
H2 = 16
NUM_SUM = 100
NUM_LEAF = 20
Y_DIMS = 16

NC = 2
NS = 16
NW = NC * NS
CHUNK = 125
ROWS = E // CHUNK
ROWS_W = ROWS // NW

S_BIG = 632
S_LAST = N - (NS - 1) * S_BIG

W_P0 = 128
W_P1 = 128
W_DEG = 8

RECON_BLK = 400
TC_BLK = 1000
LEAF_W = Y_DIMS * NUM_LEAF
OUT_WORDS = N * N + N * NUM_SUM + N * Y_DIMS * NUM_LEAF


def _sc_mesh():
    return plsc.VectorSubcoreMesh(core_axis_name="c", subcore_axis_name="s")


def _stripe_zero(zer_hbm, acc, s):
    @pl.when(s < NS - 1)
    def _():
        pltpu.sync_copy(zer_hbm, acc.at[pl.ds(s * S_BIG, S_BIG)])

    @pl.when(s == NS - 1)
    def _():
        pltpu.sync_copy(zer_hbm.at[pl.ds(0, S_LAST)],
                        acc.at[pl.ds((NS - 1) * S_BIG, S_LAST)])


def _stripe_out(acc, out_hbm, c, s):
    @pl.when(s < NS - 1)
    def _():
        pltpu.sync_copy(acc.at[pl.ds(s * S_BIG, S_BIG)],
                        out_hbm.at[c, pl.ds(s * S_BIG, S_BIG)])

    @pl.when(s == NS - 1)
    def _():
        pltpu.sync_copy(acc.at[pl.ds((NS - 1) * S_BIG, S_LAST)],
                        out_hbm.at[c, pl.ds((NS - 1) * S_BIG, S_LAST)])


def _deg_kernel(dst_hbm, ones_hbm, zer_hbm, out_hbm, dst_v, ones_v, acc):
    c = lax.axis_index("c")
    s = lax.axis_index("s")
    wid = c * NS + s
    _stripe_zero(zer_hbm, acc, s)
    pltpu.sync_copy(dst_hbm.at[pl.ds(wid * ROWS_W, ROWS_W)], dst_v)
    pltpu.sync_copy(ones_hbm, ones_v)
    plsc.subcore_barrier()

    def body(j, carry):
        pltpu.sync_copy(ones_v, acc.at[dst_v.at[j]], add=True)
        return carry

    lax.fori_loop(0, ROWS_W, body, 0)
    plsc.subcore_barrier()
    _stripe_out(acc, out_hbm, c, s)


def _make_deg():
    return functools.partial(
        pl.kernel,
        mesh=_sc_mesh(),
        out_type=jax.ShapeDtypeStruct((NC, N, W_DEG), jnp.float32),
        scratch_types=[
            pltpu.VMEM((ROWS_W, CHUNK), jnp.int32),
            pltpu.VMEM((CHUNK, W_DEG), jnp.float32),
            pltpu.VMEM_SHARED((N, W_DEG), jnp.float32),
        ],
    )(_deg_kernel)


def _prop_kernel(ei_hbm, tbl_hbm, zer_hbm, out_hbm,
                 ia, ib, rows_a, rows_b, acc, si_a, si_b, sg_a, sg_b):
    c = lax.axis_index("c")
    s = lax.axis_index("s")
    wid = c * NS + s
    base = wid * ROWS_W
    _stripe_zero(zer_hbm, acc, s)
    plsc.subcore_barrier()

    pltpu.sync_copy(ei_hbm.at[base], ia)
    pltpu.async_copy(tbl_hbm.at[ia.at[0]], rows_a, sg_a)
    pltpu.async_copy(ei_hbm.at[base + 1], ib, si_b)

    def body(k, carry):
        j0 = 2 * k
        j1 = 2 * k + 1
        pltpu.make_async_copy(tbl_hbm.at[ia.at[0]], rows_a, sg_a).wait()
        pltpu.make_async_copy(ei_hbm.at[base + j1], ib, si_b).wait()
        pltpu.async_copy(tbl_hbm.at[ib.at[0]], rows_b, sg_b)
        pltpu.sync_copy(rows_a, acc.at[ia.at[1]], add=True)

        @pl.when(j0 + 2 < ROWS_W)
        def _():
            pltpu.async_copy(ei_hbm.at[base + j0 + 2], ia, si_a)

        pltpu.make_async_copy(tbl_hbm.at[ib.at[0]], rows_b, sg_b).wait()

        @pl.when(j0 + 2 < ROWS_W)
        def _():
            pltpu.make_async_copy(ei_hbm.at[base + j0 + 2], ia, si_a).wait()
            pltpu.async_copy(tbl_hbm.at[ia.at[0]], rows_a, sg_a)

        pltpu.sync_copy(rows_b, acc.at[ib.at[1]], add=True)

        @pl.when(j1 + 2 < ROWS_W)
        def _():
            pltpu.async_copy(ei_hbm.at[base + j1 + 2], ib, si_b)

        return carry

    lax.fori_loop(0, ROWS_W // 2, body, 0)
    plsc.subcore_barrier()
    _stripe_out(acc, out_hbm, c, s)


def _make_prop(width):
    return functools.partial(
        pl.kernel,
        mesh=_sc_mesh(),
        out_type=jax.ShapeDtypeStruct((NC, N, width), jnp.float32),
        scratch_types=[
            pltpu.VMEM((2, CHUNK), jnp.int32),
            pltpu.VMEM((2, CHUNK), jnp.int32),
            pltpu.VMEM((CHUNK, width), jnp.float32),
            pltpu.VMEM((CHUNK, width), jnp.float32),
            pltpu.VMEM_SHARED((N, width), jnp.float32),
            pltpu.SemaphoreType.DMA,
            pltpu.SemaphoreType.DMA,
            pltpu.SemaphoreType.DMA,
            pltpu.SemaphoreType.DMA,
        ],
    )(_prop_kernel)


def _prep_body(deg_ref, x_ref, xs_ref, dinv_ref):
    deg = deg_ref[0, :, 0:1] + deg_ref[1, :, 0:1] + 1.0
    dv = lax.rsqrt(deg)
    dinv_ref[...] = dv
    xs_ref[...] = x_ref[...] * dv


def _mid_body(p_ref, dinv_ref, wg_ref, wb_ref, out_ref):
    dv = dinv_ref[...]
    agg = (p_ref[0] + p_ref[1]) * dv
    h = jnp.maximum(
        jnp.dot(agg, wg_ref[...], preferred_element_type=jnp.float32), 0.0)
    g = jnp.dot(h, wb_ref[...], preferred_element_type=jnp.float32)
    out_ref[...] = g * dv


def _finalize_body(p_ref, dinv_ref, eps_ref, wsw_ref, bsw_ref, wl1_ref,
                   bl1_ref, wl2_ref, bl2_ref, wlw_ref, blw_ref,
                   z_ref, sw_ref, lf_ref):
    dv = dinv_ref[...]
    p = (p_ref[0] + p_ref[1]) * dv
    zm = p[:, 0:H2]
    acc = zm
    for i in range(5):
        acc = acc + eps_ref[i] * jnp.exp(p[:, H2 + H2 * i:2 * H2 + H2 * i])
    z_ref[...] = 0.2 * acc
    sw_ref[...] = (
        jnp.dot(zm, wsw_ref[...], preferred_element_type=jnp.float32)
        + bsw_ref[...])
    l1 = jnp.maximum(
        jnp.dot(zm, wl1_ref[...], preferred_element_type=jnp.float32)
        + bl1_ref[...], 0.0)
    l2 = jnp.maximum(
        jnp.dot(l1, wl2_ref[...], preferred_element_type=jnp.float32)
        + bl2_ref[...], 0.0)
    for j in range(Y_DIMS):
        lf_ref[:, pl.ds(j * NUM_LEAF, NUM_LEAF)] = (
            jnp.dot(l2[:, j * H2:(j + 1) * H2], wlw_ref[...],
                    preferred_element_type=jnp.float32) + blw_ref[...])


def _recon_body(z_ref, zt_ref, out_ref):
    out_ref[...] = jnp.dot(z_ref[...], zt_ref[...],
                           preferred_element_type=jnp.float32)


def kernel(x, edge_index, params, eps):
    f32 = jnp.float32
    src2 = edge_index[0].reshape(ROWS, CHUNK)
    dst2 = edge_index[1].reshape(ROWS, CHUNK)
    ei3 = jnp.stack([src2, dst2], axis=1)
    ones8 = jnp.ones((CHUNK, W_DEG), f32)
    zer8 = jnp.zeros((S_BIG, W_DEG), f32)
    zer128 = jnp.zeros((S_BIG, W_P0), f32)

    wg = jnp.concatenate([params['W_gc%d' % i] for i in range(5)], axis=1)
    wm = jnp.concatenate([params['W_mean%d' % i] for i in range(5)], axis=0)
    ws = jnp.zeros((5 * H1, 5 * H2), f32)
    for i in range(5):
        ws = ws.at[i * H1:(i + 1) * H1, i * H2:(i + 1) * H2].set(
            params['W_std%d' % i])
    wb = jnp.concatenate([wm, ws, jnp.zeros((5 * H1, 32), f32)], axis=1)
    wsw = params['W_sw1'] @ params['W_sw2']
    bsw = params['b_sw1'] @ params['W_sw2'] + params['b_sw2']

    deg8 = _make_deg()(dst2, ones8, zer8)

    xs, dinv = pl.pallas_call(
        _prep_body,
        grid=(N // TC_BLK,),
        in_specs=[
            pl.BlockSpec((NC, TC_BLK, W_DEG), lambda i: (0, i, 0)),
            pl.BlockSpec((TC_BLK, D_FEAT), lambda i: (i, 0)),
        ],
        out_specs=(pl.BlockSpec((TC_BLK, D_FEAT), lambda i: (i, 0)),
                   pl.BlockSpec((TC_BLK, 1), lambda i: (i, 0))),
        out_shape=(jax.ShapeDtypeStruct((N, D_FEAT), f32),
                   jax.ShapeDtypeStruct((N, 1), f32)),
    )(deg8, x)

    p0 = _make_prop(W_P0)(ei3, xs, zer128)

    gs = pl.pallas_call(
        _mid_body,
        grid=(N // TC_BLK,),
        in_specs=[
            pl.BlockSpec((NC, TC_BLK, W_P0), lambda i: (0, i, 0)),
            pl.BlockSpec((TC_BLK, 1), lambda i: (i, 0)),
            pl.BlockSpec((D_FEAT, 5 * H1), lambda i: (0, 0)),
            pl.BlockSpec((5 * H1, W_P1), lambda i: (0, 0)),
        ],
        out_specs=pl.BlockSpec((TC_BLK, W_P1), lambda i: (i, 0)),
        out_shape=jax.ShapeDtypeStruct((N, W_P1), f32),
    )(p0, dinv, wg, wb)

    p1 = _make_prop(W_P1)(ei3, gs, zer128)

    z, sw, leaf = pl.pallas_call(
        _finalize_body,
        grid=(N // TC_BLK,),
        in_specs=[
            pl.BlockSpec((NC, TC_BLK, W_P1), lambda i: (0, i, 0)),
            pl.BlockSpec((TC_BLK, 1), lambda i: (i, 0)),
            pl.BlockSpec((5, TC_BLK, H2), lambda i: (0, i, 0)),
            pl.BlockSpec((H2, NUM_SUM), lambda i: (0, 0)),
            pl.BlockSpec((1, NUM_SUM), lambda i: (0, 0)),
            pl.BlockSpec((H2, Y_DIMS * H2), lambda i: (0, 0)),
            pl.BlockSpec((1, Y_DIMS * H2), lambda i: (0, 0)),
            pl.BlockSpec((Y_DIMS * H2, Y_DIMS * H2), lambda i: (0, 0)),
            pl.BlockSpec((1, Y_DIMS * H2), lambda i: (0, 0)),
            pl.BlockSpec((H2, NUM_LEAF), lambda i: (0, 0)),
            pl.BlockSpec((1, NUM_LEAF), lambda i: (0, 0)),
        ],
        out_specs=(pl.BlockSpec((TC_BLK, H2), lambda i: (i, 0)),
                   pl.BlockSpec((TC_BLK, NUM_SUM), lambda i: (i, 0)),
                   pl.BlockSpec((TC_BLK, LEAF_W), lambda i: (i, 0))),
        out_shape=(jax.ShapeDtypeStruct((N, H2), f32),
                   jax.ShapeDtypeStruct((N, NUM_SUM), f32),
                   jax.ShapeDtypeStruct((N, LEAF_W), f32)),
    )(p1, dinv, eps, wsw, bsw.reshape(1, NUM_SUM),
      params['W_l1'], params['b_l1'].reshape(1, -1),
      params['W_l2'], params['b_l2'].reshape(1, -1),
      params['W_lw'], params['b_lw'].reshape(1, NUM_LEAF))

    zt = z.T

    recon2d = pl.pallas_call(
        _recon_body,
        grid=(N // RECON_BLK,),
        in_specs=[
            pl.BlockSpec((RECON_BLK, H2), lambda i: (i, 0)),
            pl.BlockSpec((H2, N), lambda i: (0, 0)),
        ],
        out_specs=pl.BlockSpec((RECON_BLK, N), lambda i: (i, 0)),
        out_shape=jax.ShapeDtypeStruct((OUT_WORDS // N, N), f32),
    )(z, zt)

    flat = recon2d.reshape(-1)
    flat = lax.dynamic_update_slice(flat, sw.reshape(-1), (N * N,))
    flat = lax.dynamic_update_slice(
        flat, leaf.reshape(-1), (N * N + N * NUM_SUM,))
    return flat
